# baseline (speedup 1.0000x reference)
import functools

import jax
import jax.numpy as jnp
from jax import lax
from jax.experimental import pallas as pl
from jax.experimental.pallas import tpu as pltpu
from jax.experimental.pallas import tpu_sc as plsc

N = 10000
E = 320000
D_IN = 128
H = 64
G = 16
B = 16

NC = 2
NS = 16
NW = NC * NS
EPW = 10240
EPAD = NW * EPW
RPS = 632
ACC_ROWS = NS * RPS

_F = jnp.float32
_BF = jnp.bfloat16

_sc_mesh = plsc.VectorSubcoreMesh(core_axis_name="c", subcore_axis_name="s")


def _make_segsum(D, EPG, epw0, epw1):
    gpw0 = epw0 // EPG
    gpw1 = epw1 // EPG
    assert gpw0 % 2 == 0 and gpw1 % 2 == 0 and gpw0 * EPG == epw0
    assert gpw1 * EPG == epw1 and NS * (epw0 + epw1) == EPAD
    ncores_out = 1 if epw1 == 0 else NC

    @functools.partial(
        pl.kernel,
        out_type=jax.ShapeDtypeStruct((ncores_out, ACC_ROWS, D), _F),
        mesh=_sc_mesh,
        scratch_types=[
            [pltpu.VMEM((EPG,), jnp.int32) for _ in range(4)],
            [pltpu.VMEM((EPG, D), _F) for _ in range(2)],
            pltpu.VMEM_SHARED((ACC_ROWS, D), _F),
            [pltpu.SemaphoreType.DMA for _ in range(2)],
        ],
        compiler_params=pltpu.CompilerParams(use_tc_tiling_on_sc=False),
    )
    def _sc_segsum(y_hbm, src_hbm, dst_hbm, zero_hbm, out_hbm,
                   idx, rows, acc, sems):
        sidxA, didxA, sidxB, didxB = idx
        rowsA, rowsB = rows
        semA, semB = sems
        cid = lax.axis_index("c")
        sid = lax.axis_index("s")
        pl.when(cid < ncores_out)(
            lambda: pltpu.sync_copy(zero_hbm, acc.at[pl.ds(sid * RPS, RPS)]))
        plsc.subcore_barrier()

        def wait(rbuf, sem):
            pltpu.make_async_copy(y_hbm.at[sidxA], rbuf, sem).wait()

        def run(tile_base, gpw):
            def load(grp, sbuf, dbuf):
                base = tile_base + grp * EPG
                pltpu.sync_copy(src_hbm.at[pl.ds(base, EPG)], sbuf)
                pltpu.sync_copy(dst_hbm.at[pl.ds(base, EPG)], dbuf)

            load(0, sidxA, didxA)
            pltpu.async_copy(y_hbm.at[sidxA], rowsA, semA)

            def body(i, carry):
                g = 2 * i
                load(g + 1, sidxB, didxB)
                pltpu.async_copy(y_hbm.at[sidxB], rowsB, semB)
                wait(rowsA, semA)
                pltpu.sync_copy(rowsA, acc.at[didxA], add=True)
                load(jnp.minimum(g + 2, gpw - 1), sidxA, didxA)
                pltpu.async_copy(y_hbm.at[sidxA], rowsA, semA)
                wait(rowsB, semB)
                pltpu.sync_copy(rowsB, acc.at[didxB], add=True)
                return carry

            lax.fori_loop(0, gpw // 2, body, 0)
            wait(rowsA, semA)

        pl.when(cid == 0)(lambda: run(sid * epw0, gpw0))
        if epw1:
            pl.when(cid == 1)(lambda: run(NS * epw0 + sid * epw1, gpw1))
        plsc.subcore_barrier()
        pl.when(cid < ncores_out)(
            lambda: pltpu.sync_copy(acc.at[pl.ds(sid * RPS, RPS)],
                                    out_hbm.at[cid, pl.ds(sid * RPS, RPS)]))

    return _sc_segsum


_segsum_x = _make_segsum(D_IN, 160, 20480, 0)
_segsum_h = _make_segsum(H, 512, 20480, 0)


def _dot_bf16(a, b):
    return jnp.dot(a.astype(_BF), b.astype(_BF), preferred_element_type=_F)


def _mlp1_body(x_ref, p_ref, w1_ref, b1_ref, w2_ref, b2_ref, o_ref):
    z = x_ref[...] + p_ref[0, :N]
    u = jnp.maximum(_dot_bf16(z, w1_ref[...]) + b1_ref[...], 0.0)
    o_ref[...] = jnp.maximum(_dot_bf16(u, w2_ref[...]) + b2_ref[...], 0.0)


def _final_body(h1_ref, q_ref, w3_ref, b3_ref, w4_ref, b4_ref, bidx_ref,
                wih_t_ref, bih_ref, bhh_ref, wout_t_ref, bout_ref,
                len_ref, o_ref):
    z2 = h1_ref[...] + q_ref[0, :N]
    u = jnp.maximum(_dot_bf16(z2, w3_ref[...]) + b3_ref[...], 0.0)
    h2 = jnp.maximum(_dot_bf16(u, w4_ref[...]) + b4_ref[...], 0.0)
    seg = lax.broadcasted_iota(jnp.int32, (G, N), 0)
    onehot = (seg == jnp.broadcast_to(bidx_ref[...], (G, N))).astype(_F)
    sums = jnp.dot(onehot, h2, preferred_element_type=_F,
                   precision=lax.Precision.HIGHEST)
    counts = jnp.sum(onehot, axis=1, keepdims=True)
    g = sums / jnp.maximum(counts, 1.0)
    gi = _dot_bf16(g, wih_t_ref[...]) + bih_ref[...]
    gh = jnp.broadcast_to(bhh_ref[...], (B, 3 * H))
    r = jax.nn.sigmoid(gi[:, :H] + gh[:, :H])
    zz = jax.nn.sigmoid(gi[:, H:2 * H] + gh[:, H:2 * H])
    nn_ = jnp.tanh(gi[:, 2 * H:] + r * gh[:, 2 * H:])
    h_new = (1.0 - zz) * nn_
    m = jnp.reshape(len_ref[...], (B, 1)) > 0
    h_last = jnp.where(m, h_new, 0.0)
    o_ref[...] = _dot_bf16(h_last, wout_t_ref[...]) + bout_ref[...]


def kernel(x, edge_index, batch_idx, lengths, W1, b1, W2, b2, W3, b3, W4, b4,
           W_ih, b_ih, W_hh, b_hh, W_out, b_out):
    pad = EPAD - E
    src2 = jnp.concatenate([edge_index[0], jnp.zeros((pad,), jnp.int32)])
    dst2 = jnp.concatenate([edge_index[1], jnp.full((pad,), N, jnp.int32)])
    zeros_x = jnp.zeros((RPS, D_IN), _F)
    zeros_h = jnp.zeros((RPS, H), _F)

    p = _segsum_x(x, src2, dst2, zeros_x)

    h1 = pl.pallas_call(
        _mlp1_body, out_shape=jax.ShapeDtypeStruct((N, H), _F))(
            x, p, W1, b1.reshape(1, H), W2, b2.reshape(1, H))

    q = _segsum_h(h1, src2, dst2, zeros_h)

    out = pl.pallas_call(
        _final_body, out_shape=jax.ShapeDtypeStruct((B, 1), _F))(
            h1, q, W3, b3.reshape(1, H), W4, b4.reshape(1, H),
            batch_idx.reshape(1, N), W_ih.T, b_ih.reshape(1, 3 * H),
            b_hh.reshape(1, 3 * H), W_out.T, b_out.reshape(1, 1),
            lengths.reshape(1, B))
    return out[:, 0]


# speedup vs baseline: 1.3536x; 1.3536x over previous
import functools

import jax
import jax.numpy as jnp
from jax import lax
from jax.experimental import pallas as pl
from jax.experimental.pallas import tpu as pltpu
from jax.experimental.pallas import tpu_sc as plsc

N = 10000
E = 320000
D_IN = 128
H = 64
G = 16
B = 16

NC = 2
NS = 16
NW = NC * NS
EPW = 10240
EPAD = NW * EPW
RPS = 632
ACC_ROWS = NS * RPS

_F = jnp.float32
_BF = jnp.bfloat16

_sc_mesh = plsc.VectorSubcoreMesh(core_axis_name="c", subcore_axis_name="s")


def _make_segsum(D, EPG, epw0, epw1, stage=False):
    gpw0 = epw0 // EPG
    gpw1 = epw1 // EPG
    assert gpw0 % 2 == 0 and gpw1 % 2 == 0 and gpw0 * EPG == epw0
    assert gpw1 * EPG == epw1 and NS * (epw0 + epw1) == EPAD
    ncores_out = 1 if epw1 == 0 else NC

    @functools.partial(
        pl.kernel,
        out_type=jax.ShapeDtypeStruct((ncores_out, ACC_ROWS, D), _F),
        mesh=_sc_mesh,
        scratch_types=[
            [pltpu.VMEM((EPG,), jnp.int32) for _ in range(4)],
            [pltpu.VMEM((EPG, D), _F) for _ in range(2)],
            pltpu.VMEM_SHARED((ACC_ROWS, D), _F),
            [pltpu.VMEM_SHARED((ACC_ROWS, D), _F) for _ in range(1 if stage else 0)],
            [pltpu.SemaphoreType.DMA for _ in range(2)],
        ],
        compiler_params=pltpu.CompilerParams(use_tc_tiling_on_sc=False),
    )
    def _sc_segsum(y_hbm, src_hbm, dst_hbm, zero_hbm, out_hbm,
                   idx, rows, acc, tbl, sems):
        sidxA, didxA, sidxB, didxB = idx
        rowsA, rowsB = rows
        semA, semB = sems
        cid = lax.axis_index("c")
        sid = lax.axis_index("s")
        pl.when(cid < ncores_out)(
            lambda: pltpu.sync_copy(zero_hbm, acc.at[pl.ds(sid * RPS, RPS)]))
        if stage:
            pltpu.sync_copy(y_hbm.at[pl.ds(sid * RPS, RPS)],
                            tbl[0].at[pl.ds(sid * RPS, RPS)])
            table = tbl[0]
        else:
            table = y_hbm
        plsc.subcore_barrier()

        def wait(rbuf, sem):
            pltpu.make_async_copy(table.at[sidxA], rbuf, sem).wait()

        def run(tile_base, gpw):
            def load(grp, sbuf, dbuf):
                base = tile_base + grp * EPG
                pltpu.sync_copy(src_hbm.at[pl.ds(base, EPG)], sbuf)
                pltpu.sync_copy(dst_hbm.at[pl.ds(base, EPG)], dbuf)

            load(0, sidxA, didxA)
            pltpu.async_copy(table.at[sidxA], rowsA, semA)

            def body(i, carry):
                g = 2 * i
                load(g + 1, sidxB, didxB)
                pltpu.async_copy(table.at[sidxB], rowsB, semB)
                wait(rowsA, semA)
                pltpu.sync_copy(rowsA, acc.at[didxA], add=True)
                load(jnp.minimum(g + 2, gpw - 1), sidxA, didxA)
                pltpu.async_copy(table.at[sidxA], rowsA, semA)
                wait(rowsB, semB)
                pltpu.sync_copy(rowsB, acc.at[didxB], add=True)
                return carry

            lax.fori_loop(0, gpw // 2, body, 0)
            wait(rowsA, semA)

        pl.when(cid == 0)(lambda: run(sid * epw0, gpw0))
        if epw1:
            pl.when(cid == 1)(lambda: run(NS * epw0 + sid * epw1, gpw1))
        plsc.subcore_barrier()
        pl.when(cid < ncores_out)(
            lambda: pltpu.sync_copy(acc.at[pl.ds(sid * RPS, RPS)],
                                    out_hbm.at[cid, pl.ds(sid * RPS, RPS)]))

    return _sc_segsum


_segsum_x = _make_segsum(D_IN, 160, 15360, 5120)
_segsum_h = _make_segsum(H, 256, 10240, 10240, stage=True)


def _dot_bf16(a, b):
    return jnp.dot(a.astype(_BF), b.astype(_BF), preferred_element_type=_F)


def _mlp1_body(x_ref, p_ref, w1_ref, b1_ref, w2_ref, b2_ref, o_ref):
    z = x_ref[...] + (p_ref[0, :N] + p_ref[1, :N])
    u = jnp.maximum(_dot_bf16(z, w1_ref[...]) + b1_ref[...], 0.0)
    o_ref[:N] = jnp.maximum(_dot_bf16(u, w2_ref[...]) + b2_ref[...], 0.0)
    o_ref[N:] = jnp.zeros((ACC_ROWS - N, H), _F)


def _final_body(h1_ref, q_ref, w3_ref, b3_ref, w4_ref, b4_ref, bidx_ref,
                wih_t_ref, bih_ref, bhh_ref, wout_t_ref, bout_ref,
                len_ref, o_ref):
    z2 = h1_ref[:N] + (q_ref[0, :N] + q_ref[1, :N])
    u = jnp.maximum(_dot_bf16(z2, w3_ref[...]) + b3_ref[...], 0.0)
    h2 = jnp.maximum(_dot_bf16(u, w4_ref[...]) + b4_ref[...], 0.0)
    seg = lax.broadcasted_iota(jnp.int32, (G, N), 0)
    onehot = (seg == jnp.broadcast_to(bidx_ref[...], (G, N))).astype(_F)
    sums = jnp.dot(onehot, h2, preferred_element_type=_F,
                   precision=lax.Precision.HIGHEST)
    counts = jnp.sum(onehot, axis=1, keepdims=True)
    g = sums / jnp.maximum(counts, 1.0)
    gi = _dot_bf16(g, wih_t_ref[...]) + bih_ref[...]
    gh = jnp.broadcast_to(bhh_ref[...], (B, 3 * H))
    r = jax.nn.sigmoid(gi[:, :H] + gh[:, :H])
    zz = jax.nn.sigmoid(gi[:, H:2 * H] + gh[:, H:2 * H])
    nn_ = jnp.tanh(gi[:, 2 * H:] + r * gh[:, 2 * H:])
    h_new = (1.0 - zz) * nn_
    m = jnp.reshape(len_ref[...], (B, 1)) > 0
    h_last = jnp.where(m, h_new, 0.0)
    o_ref[...] = _dot_bf16(h_last, wout_t_ref[...]) + bout_ref[...]


def kernel(x, edge_index, batch_idx, lengths, W1, b1, W2, b2, W3, b3, W4, b4,
           W_ih, b_ih, W_hh, b_hh, W_out, b_out):
    pad = EPAD - E
    src2 = jnp.concatenate([edge_index[0], jnp.zeros((pad,), jnp.int32)])
    dst2 = jnp.concatenate([edge_index[1], jnp.full((pad,), N, jnp.int32)])
    zeros_x = jnp.zeros((RPS, D_IN), _F)
    zeros_h = jnp.zeros((RPS, H), _F)

    p = _segsum_x(x, src2, dst2, zeros_x)

    h1 = pl.pallas_call(
        _mlp1_body, out_shape=jax.ShapeDtypeStruct((ACC_ROWS, H), _F))(
            x, p, W1, b1.reshape(1, H), W2, b2.reshape(1, H))

    q = _segsum_h(h1, src2, dst2, zeros_h)

    out = pl.pallas_call(
        _final_body, out_shape=jax.ShapeDtypeStruct((B, 1), _F))(
            h1, q, W3, b3.reshape(1, H), W4, b4.reshape(1, H),
            batch_idx.reshape(1, N), W_ih.T, b_ih.reshape(1, 3 * H),
            b_hh.reshape(1, 3 * H), W_out.T, b_out.reshape(1, 1),
            lengths.reshape(1, B))
    return out[:, 0]


# speedup vs baseline: 2.0846x; 1.5400x over previous
import functools

import jax
import jax.numpy as jnp
from jax import lax
from jax.experimental import pallas as pl
from jax.experimental.pallas import tpu as pltpu
from jax.experimental.pallas import tpu_sc as plsc

N = 10000
E = 320000
D_IN = 128
H = 64
G = 16
B = 16

NC = 2
NS = 16
NW = NC * NS
EPW = 10240
EPAD = NW * EPW
RPS = 632
ACC_ROWS = NS * RPS

_F = jnp.float32
_BF = jnp.bfloat16

_sc_mesh = plsc.VectorSubcoreMesh(core_axis_name="c", subcore_axis_name="s")


def _make_segsum(D, EPG, epw0, epw1, stage=False):
    gpw0 = epw0 // EPG
    gpw1 = epw1 // EPG
    assert gpw0 % 2 == 0 and gpw1 % 2 == 0 and gpw0 * EPG == epw0
    assert gpw1 * EPG == epw1 and NS * (epw0 + epw1) == EPAD
    ncores_out = 1 if epw1 == 0 else NC

    @functools.partial(
        pl.kernel,
        out_type=jax.ShapeDtypeStruct((ncores_out, ACC_ROWS, D), _F),
        mesh=_sc_mesh,
        scratch_types=[
            [pltpu.VMEM((EPG,), jnp.int32) for _ in range(4)],
            [pltpu.VMEM((EPG, D), _F) for _ in range(2)],
            pltpu.VMEM_SHARED((ACC_ROWS, D), _F),
            [pltpu.VMEM_SHARED((ACC_ROWS, D), _F) for _ in range(1 if stage else 0)],
            [pltpu.SemaphoreType.DMA for _ in range(2)],
        ],
        compiler_params=pltpu.CompilerParams(use_tc_tiling_on_sc=False),
    )
    def _sc_segsum(y_hbm, src_hbm, dst_hbm, zero_hbm, out_hbm,
                   idx, rows, acc, tbl, sems):
        sidxA, didxA, sidxB, didxB = idx
        rowsA, rowsB = rows
        semA, semB = sems
        cid = lax.axis_index("c")
        sid = lax.axis_index("s")
        pl.when(cid < ncores_out)(
            lambda: pltpu.sync_copy(zero_hbm, acc.at[pl.ds(sid * RPS, RPS)]))
        if stage:
            pltpu.sync_copy(y_hbm.at[pl.ds(sid * RPS, RPS)],
                            tbl[0].at[pl.ds(sid * RPS, RPS)])
            table = tbl[0]
        else:
            table = y_hbm
        plsc.subcore_barrier()

        def wait(rbuf, sem):
            pltpu.make_async_copy(table.at[sidxA], rbuf, sem).wait()

        def run(tile_base, gpw):
            def load(grp, sbuf, dbuf):
                base = tile_base + grp * EPG
                pltpu.sync_copy(src_hbm.at[pl.ds(base, EPG)], sbuf)
                pltpu.sync_copy(dst_hbm.at[pl.ds(base, EPG)], dbuf)

            load(0, sidxA, didxA)
            pltpu.async_copy(table.at[sidxA], rowsA, semA)

            def body(i, carry):
                g = 2 * i
                load(g + 1, sidxB, didxB)
                pltpu.async_copy(table.at[sidxB], rowsB, semB)
                wait(rowsA, semA)
                pltpu.sync_copy(rowsA, acc.at[didxA], add=True)
                load(jnp.minimum(g + 2, gpw - 1), sidxA, didxA)
                pltpu.async_copy(table.at[sidxA], rowsA, semA)
                wait(rowsB, semB)
                pltpu.sync_copy(rowsB, acc.at[didxB], add=True)
                return carry

            lax.fori_loop(0, gpw // 2, body, 0)
            wait(rowsA, semA)

        pl.when(cid == 0)(lambda: run(sid * epw0, gpw0))
        if epw1:
            pl.when(cid == 1)(lambda: run(NS * epw0 + sid * epw1, gpw1))
        plsc.subcore_barrier()
        pl.when(cid < ncores_out)(
            lambda: pltpu.sync_copy(acc.at[pl.ds(sid * RPS, RPS)],
                                    out_hbm.at[cid, pl.ds(sid * RPS, RPS)]))

    return _sc_segsum


def _make_segsum_colsplit(EPG):
    ept = EPAD // NS
    gpw = ept // EPG
    assert gpw % 2 == 0
    D = H

    @functools.partial(
        pl.kernel,
        out_type=jax.ShapeDtypeStruct((NC, ACC_ROWS, D), _F),
        mesh=_sc_mesh,
        scratch_types=[
            [pltpu.VMEM((EPG,), jnp.int32) for _ in range(4)],
            [pltpu.VMEM((EPG, D), _F) for _ in range(2)],
            pltpu.VMEM_SHARED((ACC_ROWS, D), _F),
            pltpu.VMEM_SHARED((ACC_ROWS, D), _F),
            [pltpu.SemaphoreType.DMA for _ in range(2)],
        ],
        compiler_params=pltpu.CompilerParams(use_tc_tiling_on_sc=False),
    )
    def _sc_segsum_cs(xl_hbm, xr_hbm, src_hbm, dst_hbm, zero_hbm, out_hbm,
                      idx, rows, acc, table, sems):
        sidxA, didxA, sidxB, didxB = idx
        rowsA, rowsB = rows
        semA, semB = sems
        cid = lax.axis_index("c")
        sid = lax.axis_index("s")
        pltpu.sync_copy(zero_hbm, acc.at[pl.ds(sid * RPS, RPS)])
        pl.when(cid == 0)(
            lambda: pltpu.sync_copy(xl_hbm.at[pl.ds(sid * RPS, RPS)],
                                    table.at[pl.ds(sid * RPS, RPS)]))
        pl.when(cid == 1)(
            lambda: pltpu.sync_copy(xr_hbm.at[pl.ds(sid * RPS, RPS)],
                                    table.at[pl.ds(sid * RPS, RPS)]))
        plsc.subcore_barrier()

        def wait(rbuf, sem):
            pltpu.make_async_copy(table.at[sidxA], rbuf, sem).wait()

        def load(grp, sbuf, dbuf):
            base = sid * ept + grp * EPG
            pltpu.sync_copy(src_hbm.at[pl.ds(base, EPG)], sbuf)
            pltpu.sync_copy(dst_hbm.at[pl.ds(base, EPG)], dbuf)

        load(0, sidxA, didxA)
        pltpu.async_copy(table.at[sidxA], rowsA, semA)

        def body(i, carry):
            g = 2 * i
            load(g + 1, sidxB, didxB)
            pltpu.async_copy(table.at[sidxB], rowsB, semB)
            wait(rowsA, semA)
            pltpu.sync_copy(rowsA, acc.at[didxA], add=True)
            load(jnp.minimum(g + 2, gpw - 1), sidxA, didxA)
            pltpu.async_copy(table.at[sidxA], rowsA, semA)
            wait(rowsB, semB)
            pltpu.sync_copy(rowsB, acc.at[didxB], add=True)
            return carry

        lax.fori_loop(0, gpw // 2, body, 0)
        wait(rowsA, semA)
        plsc.subcore_barrier()
        pltpu.sync_copy(acc.at[pl.ds(sid * RPS, RPS)],
                        out_hbm.at[cid, pl.ds(sid * RPS, RPS)])

    return _sc_segsum_cs


_segsum_x = _make_segsum_colsplit(256)
_segsum_h = _make_segsum(H, 256, 10240, 10240, stage=True)


def _dot_bf16(a, b):
    return jnp.dot(a.astype(_BF), b.astype(_BF), preferred_element_type=_F)


def _mlp1_body(x_ref, p_ref, w1_ref, b1_ref, w2_ref, b2_ref, o_ref):
    z = x_ref[...] + jnp.concatenate([p_ref[0, :N], p_ref[1, :N]], axis=1)
    u = jnp.maximum(_dot_bf16(z, w1_ref[...]) + b1_ref[...], 0.0)
    o_ref[:N] = jnp.maximum(_dot_bf16(u, w2_ref[...]) + b2_ref[...], 0.0)
    o_ref[N:] = jnp.zeros((ACC_ROWS - N, H), _F)


def _final_body(h1_ref, q_ref, w3_ref, b3_ref, w4_ref, b4_ref, bidx_ref,
                wih_t_ref, bih_ref, bhh_ref, wout_t_ref, bout_ref,
                len_ref, o_ref):
    z2 = h1_ref[:N] + (q_ref[0, :N] + q_ref[1, :N])
    u = jnp.maximum(_dot_bf16(z2, w3_ref[...]) + b3_ref[...], 0.0)
    h2 = jnp.maximum(_dot_bf16(u, w4_ref[...]) + b4_ref[...], 0.0)
    seg = lax.broadcasted_iota(jnp.int32, (G, N), 0)
    onehot = (seg == jnp.broadcast_to(bidx_ref[...], (G, N))).astype(_F)
    sums = jnp.dot(onehot, h2, preferred_element_type=_F,
                   precision=lax.Precision.HIGHEST)
    counts = jnp.sum(onehot, axis=1, keepdims=True)
    g = sums / jnp.maximum(counts, 1.0)
    gi = _dot_bf16(g, wih_t_ref[...]) + bih_ref[...]
    gh = jnp.broadcast_to(bhh_ref[...], (B, 3 * H))
    r = jax.nn.sigmoid(gi[:, :H] + gh[:, :H])
    zz = jax.nn.sigmoid(gi[:, H:2 * H] + gh[:, H:2 * H])
    nn_ = jnp.tanh(gi[:, 2 * H:] + r * gh[:, 2 * H:])
    h_new = (1.0 - zz) * nn_
    m = jnp.reshape(len_ref[...], (B, 1)) > 0
    h_last = jnp.where(m, h_new, 0.0)
    o_ref[...] = _dot_bf16(h_last, wout_t_ref[...]) + bout_ref[...]


def kernel(x, edge_index, batch_idx, lengths, W1, b1, W2, b2, W3, b3, W4, b4,
           W_ih, b_ih, W_hh, b_hh, W_out, b_out):
    pad = EPAD - E
    src2 = jnp.concatenate([edge_index[0], jnp.zeros((pad,), jnp.int32)])
    dst2 = jnp.concatenate([edge_index[1], jnp.full((pad,), N, jnp.int32)])
    zeros_h = jnp.zeros((RPS, H), _F)
    xpad = jnp.zeros((ACC_ROWS - N, H), _F)
    xl = jnp.concatenate([x[:, :H], xpad])
    xr = jnp.concatenate([x[:, H:], xpad])

    p = _segsum_x(xl, xr, src2, dst2, zeros_h)

    h1 = pl.pallas_call(
        _mlp1_body, out_shape=jax.ShapeDtypeStruct((ACC_ROWS, H), _F))(
            x, p, W1, b1.reshape(1, H), W2, b2.reshape(1, H))

    q = _segsum_h(h1, src2, dst2, zeros_h)

    out = pl.pallas_call(
        _final_body, out_shape=jax.ShapeDtypeStruct((B, 1), _F))(
            h1, q, W3, b3.reshape(1, H), W4, b4.reshape(1, H),
            batch_idx.reshape(1, N), W_ih.T, b_ih.reshape(1, 3 * H),
            b_hh.reshape(1, 3 * H), W_out.T, b_out.reshape(1, 1),
            lengths.reshape(1, B))
    return out[:, 0]


# speedup vs baseline: 2.1473x; 1.0301x over previous
import functools

import jax
import jax.numpy as jnp
from jax import lax
from jax.experimental import pallas as pl
from jax.experimental.pallas import tpu as pltpu
from jax.experimental.pallas import tpu_sc as plsc

N = 10000
E = 320000
D_IN = 128
H = 64
G = 16
B = 16

NC = 2
NS = 16
NW = NC * NS
EPW = 10240
EPAD = NW * EPW
RPS = 632
ACC_ROWS = NS * RPS

_F = jnp.float32
_BF = jnp.bfloat16

_sc_mesh = plsc.VectorSubcoreMesh(core_axis_name="c", subcore_axis_name="s")


def _make_segsum(D, EPG, epw0, epw1, stage=False):
    gpw0 = epw0 // EPG
    gpw1 = epw1 // EPG
    assert gpw0 % 2 == 0 and gpw1 % 2 == 0 and gpw0 * EPG == epw0
    assert gpw1 * EPG == epw1 and NS * (epw0 + epw1) == EPAD
    ncores_out = 1 if epw1 == 0 else NC

    @functools.partial(
        pl.kernel,
        out_type=jax.ShapeDtypeStruct((ncores_out, ACC_ROWS, D), _F),
        mesh=_sc_mesh,
        scratch_types=[
            [pltpu.VMEM((EPG,), jnp.int32) for _ in range(4)],
            [pltpu.VMEM((EPG, D), _F) for _ in range(2)],
            pltpu.VMEM_SHARED((ACC_ROWS, D), _F),
            [pltpu.VMEM_SHARED((ACC_ROWS, D), _F) for _ in range(1 if stage else 0)],
            [pltpu.SemaphoreType.DMA for _ in range(2)],
        ],
        compiler_params=pltpu.CompilerParams(use_tc_tiling_on_sc=False),
    )
    def _sc_segsum(y_hbm, src_hbm, dst_hbm, zero_hbm, out_hbm,
                   idx, rows, acc, tbl, sems):
        sidxA, didxA, sidxB, didxB = idx
        rowsA, rowsB = rows
        semA, semB = sems
        cid = lax.axis_index("c")
        sid = lax.axis_index("s")
        pl.when(cid < ncores_out)(
            lambda: pltpu.sync_copy(zero_hbm, acc.at[pl.ds(sid * RPS, RPS)]))
        if stage:
            pltpu.sync_copy(y_hbm.at[pl.ds(sid * RPS, RPS)],
                            tbl[0].at[pl.ds(sid * RPS, RPS)])
            table = tbl[0]
        else:
            table = y_hbm
        plsc.subcore_barrier()

        def wait(rbuf, sem):
            pltpu.make_async_copy(table.at[sidxA], rbuf, sem).wait()

        def run(tile_base, gpw):
            def load(grp, sbuf, dbuf):
                base = tile_base + grp * EPG
                pltpu.sync_copy(src_hbm.at[pl.ds(base, EPG)], sbuf)
                pltpu.sync_copy(dst_hbm.at[pl.ds(base, EPG)], dbuf)

            load(0, sidxA, didxA)
            pltpu.async_copy(table.at[sidxA], rowsA, semA)

            def body(i, carry):
                g = 2 * i
                load(g + 1, sidxB, didxB)
                pltpu.async_copy(table.at[sidxB], rowsB, semB)
                wait(rowsA, semA)
                pltpu.sync_copy(rowsA, acc.at[didxA], add=True)
                load(jnp.minimum(g + 2, gpw - 1), sidxA, didxA)
                pltpu.async_copy(table.at[sidxA], rowsA, semA)
                wait(rowsB, semB)
                pltpu.sync_copy(rowsB, acc.at[didxB], add=True)
                return carry

            lax.fori_loop(0, gpw // 2, body, 0)
            wait(rowsA, semA)

        pl.when(cid == 0)(lambda: run(sid * epw0, gpw0))
        if epw1:
            pl.when(cid == 1)(lambda: run(NS * epw0 + sid * epw1, gpw1))
        plsc.subcore_barrier()
        pl.when(cid < ncores_out)(
            lambda: pltpu.sync_copy(acc.at[pl.ds(sid * RPS, RPS)],
                                    out_hbm.at[cid, pl.ds(sid * RPS, RPS)]))

    return _sc_segsum


def _make_segsum_colsplit(EPG):
    ept = EPAD // NS
    gpw = ept // EPG
    assert gpw % 2 == 0
    D = H

    @functools.partial(
        pl.kernel,
        out_type=jax.ShapeDtypeStruct((NC, ACC_ROWS, D), _F),
        mesh=_sc_mesh,
        scratch_types=[
            [pltpu.VMEM((EPG,), jnp.int32) for _ in range(4)],
            [pltpu.VMEM((EPG, D), _F) for _ in range(2)],
            pltpu.VMEM_SHARED((ACC_ROWS, D), _F),
            pltpu.VMEM_SHARED((ACC_ROWS, D), _F),
            [pltpu.SemaphoreType.DMA for _ in range(2)],
        ],
        compiler_params=pltpu.CompilerParams(use_tc_tiling_on_sc=False),
    )
    def _sc_segsum_cs(xl_hbm, xr_hbm, src_hbm, dst_hbm, zero_hbm, out_hbm,
                      idx, rows, acc, table, sems):
        sidxA, didxA, sidxB, didxB = idx
        rowsA, rowsB = rows
        semA, semB = sems
        cid = lax.axis_index("c")
        sid = lax.axis_index("s")
        pltpu.sync_copy(zero_hbm, acc.at[pl.ds(sid * RPS, RPS)])
        pl.when(cid == 0)(
            lambda: pltpu.sync_copy(xl_hbm.at[pl.ds(sid * RPS, RPS)],
                                    table.at[pl.ds(sid * RPS, RPS)]))
        pl.when(cid == 1)(
            lambda: pltpu.sync_copy(xr_hbm.at[pl.ds(sid * RPS, RPS)],
                                    table.at[pl.ds(sid * RPS, RPS)]))
        plsc.subcore_barrier()

        def wait(rbuf, sem):
            pltpu.make_async_copy(table.at[sidxA], rbuf, sem).wait()

        def load(grp, sbuf, dbuf):
            base = sid * ept + grp * EPG
            pltpu.sync_copy(src_hbm.at[pl.ds(base, EPG)], sbuf)
            pltpu.sync_copy(dst_hbm.at[pl.ds(base, EPG)], dbuf)

        load(0, sidxA, didxA)
        pltpu.async_copy(table.at[sidxA], rowsA, semA)

        def body(i, carry):
            g = 2 * i
            load(g + 1, sidxB, didxB)
            pltpu.async_copy(table.at[sidxB], rowsB, semB)
            wait(rowsA, semA)
            pltpu.sync_copy(rowsA, acc.at[didxA], add=True)
            load(jnp.minimum(g + 2, gpw - 1), sidxA, didxA)
            pltpu.async_copy(table.at[sidxA], rowsA, semA)
            wait(rowsB, semB)
            pltpu.sync_copy(rowsB, acc.at[didxB], add=True)
            return carry

        lax.fori_loop(0, gpw // 2, body, 0)
        wait(rowsA, semA)
        plsc.subcore_barrier()
        pltpu.sync_copy(acc.at[pl.ds(sid * RPS, RPS)],
                        out_hbm.at[cid, pl.ds(sid * RPS, RPS)])

    return _sc_segsum_cs


_segsum_x = _make_segsum_colsplit(320)
_segsum_h = _make_segsum(H, 320, 10240, 10240, stage=True)


def _dot_bf16(a, b):
    return jnp.dot(a.astype(_BF), b.astype(_BF), preferred_element_type=_F)


def _mlp1_body(x_ref, p_ref, w1_ref, b1_ref, w2_ref, b2_ref, o_ref):
    z = x_ref[...] + jnp.concatenate([p_ref[0, :N], p_ref[1, :N]], axis=1)
    u = jnp.maximum(_dot_bf16(z, w1_ref[...]) + b1_ref[...], 0.0)
    o_ref[:N] = jnp.maximum(_dot_bf16(u, w2_ref[...]) + b2_ref[...], 0.0)
    o_ref[N:] = jnp.zeros((ACC_ROWS - N, H), _F)


def _final_body(h1_ref, q_ref, w3_ref, b3_ref, w4_ref, b4_ref, bidx_ref,
                wih_t_ref, bih_ref, bhh_ref, wout_t_ref, bout_ref,
                len_ref, o_ref):
    z2 = h1_ref[:N] + (q_ref[0, :N] + q_ref[1, :N])
    u = jnp.maximum(_dot_bf16(z2, w3_ref[...]) + b3_ref[...], 0.0)
    h2 = jnp.maximum(_dot_bf16(u, w4_ref[...]) + b4_ref[...], 0.0)
    seg = lax.broadcasted_iota(jnp.int32, (G, N), 0)
    onehot = (seg == jnp.broadcast_to(bidx_ref[...], (G, N))).astype(_F)
    sums = jnp.dot(onehot, h2, preferred_element_type=_F,
                   precision=lax.Precision.HIGHEST)
    counts = jnp.sum(onehot, axis=1, keepdims=True)
    g = sums / jnp.maximum(counts, 1.0)
    gi = _dot_bf16(g, wih_t_ref[...]) + bih_ref[...]
    gh = jnp.broadcast_to(bhh_ref[...], (B, 3 * H))
    r = jax.nn.sigmoid(gi[:, :H] + gh[:, :H])
    zz = jax.nn.sigmoid(gi[:, H:2 * H] + gh[:, H:2 * H])
    nn_ = jnp.tanh(gi[:, 2 * H:] + r * gh[:, 2 * H:])
    h_new = (1.0 - zz) * nn_
    m = jnp.reshape(len_ref[...], (B, 1)) > 0
    h_last = jnp.where(m, h_new, 0.0)
    o_ref[...] = _dot_bf16(h_last, wout_t_ref[...]) + bout_ref[...]


def kernel(x, edge_index, batch_idx, lengths, W1, b1, W2, b2, W3, b3, W4, b4,
           W_ih, b_ih, W_hh, b_hh, W_out, b_out):
    pad = EPAD - E
    src2 = jnp.concatenate([edge_index[0], jnp.zeros((pad,), jnp.int32)])
    dst2 = jnp.concatenate([edge_index[1], jnp.full((pad,), N, jnp.int32)])
    zeros_h = jnp.zeros((RPS, H), _F)
    xpad = jnp.zeros((ACC_ROWS - N, H), _F)
    xl = jnp.concatenate([x[:, :H], xpad])
    xr = jnp.concatenate([x[:, H:], xpad])

    p = _segsum_x(xl, xr, src2, dst2, zeros_h)

    h1 = pl.pallas_call(
        _mlp1_body, out_shape=jax.ShapeDtypeStruct((ACC_ROWS, H), _F))(
            x, p, W1, b1.reshape(1, H), W2, b2.reshape(1, H))

    q = _segsum_h(h1, src2, dst2, zeros_h)

    out = pl.pallas_call(
        _final_body, out_shape=jax.ShapeDtypeStruct((B, 1), _F))(
            h1, q, W3, b3.reshape(1, H), W4, b4.reshape(1, H),
            batch_idx.reshape(1, N), W_ih.T, b_ih.reshape(1, 3 * H),
            b_hh.reshape(1, 3 * H), W_out.T, b_out.reshape(1, 1),
            lengths.reshape(1, B))
    return out[:, 0]
